# baseline (speedup 1.0000x reference)
import jax
import jax.numpy as jnp
from jax import lax
from jax.experimental import pallas as pl
from jax.experimental.pallas import tpu as pltpu
from jax.experimental.pallas import tpu_sc as plsc

VOCAB = 1000000
EMBED = 64
SEQ = 200
BATCH = 4096

_NC = 2
_NS = 16
_NW = _NC * _NS
_BPW = BATCH // _NW
_GRP = 8
_NGRP = _BPW // _GRP
_S0, _S1 = 104, 96


def _pool_body(x_hbm, emb_hbm, out_hbm, slab_v, idxt_v, lsb_v, rows0, rows1,
               stage_v, sem0, sem1):
    wid = lax.axis_index("s") * _NC + lax.axis_index("c")
    base0 = wid * _BPW
    inv = jnp.float32(1.0 / SEQ)
    z = jnp.zeros((16,), jnp.float32)
    bufs = (rows0, rows1)
    sems = (sem0, sem1)

    pltpu.sync_copy(x_hbm.at[:, pl.ds(base0, _BPW)], slab_v)
    ci = jnp.arange(16, dtype=jnp.int32) * SEQ

    def tr_body(r, carry):
        for e0 in range(0, _BPW, 16):
            v = slab_v[r, pl.ds(e0, 16)]
            pos = ci + (e0 * SEQ + r)
            plsc.store_scatter(idxt_v, [pos], lax.shift_right_logical(v, 1))
            plsc.store_scatter(lsb_v, [pos], (v & 1) * 64)
        return carry

    lax.fori_loop(0, SEQ, tr_body, 0)

    def fire(g, rows_v, sem):
        off = g * SEQ
        pltpu.async_copy(
            emb_hbm.at[idxt_v.at[pl.ds(off, _S0)]],
            rows_v.at[pl.ds(0, _S0), :],
            sem,
        )
        pltpu.async_copy(
            emb_hbm.at[idxt_v.at[pl.ds(off + _S0, _S1)]],
            rows_v.at[pl.ds(_S0, _S1), :],
            sem,
        )

    def drain(g, rows_v, sem):
        off = g * SEQ
        pltpu.make_async_copy(
            emb_hbm.at[idxt_v.at[pl.ds(off, _S0)]],
            rows_v.at[pl.ds(0, _S0), :],
            sem,
        ).wait()
        pltpu.make_async_copy(
            emb_hbm.at[idxt_v.at[pl.ds(off + _S0, _S1)]],
            rows_v.at[pl.ds(_S0, _S1), :],
            sem,
        ).wait()

    def accum(g, k, rows_v):
        goff = g * SEQ

        def row_body(r, acc):
            b0, b1, b2, b3, c0, c1, c2, c3 = acc
            r0 = r * 4
            hv = lsb_v[pl.ds(goff + r0, 16)]
            h0 = hv[0]
            h1 = hv[1]
            h2 = hv[2]
            h3 = hv[3]
            b0 = b0 + rows_v[r0, pl.ds(h0, 16)]
            b1 = b1 + rows_v[r0, pl.ds(h0 + 16, 16)]
            b2 = b2 + rows_v[r0, pl.ds(h0 + 32, 16)]
            b3 = b3 + rows_v[r0, pl.ds(h0 + 48, 16)]
            c0 = c0 + rows_v[r0 + 1, pl.ds(h1, 16)]
            c1 = c1 + rows_v[r0 + 1, pl.ds(h1 + 16, 16)]
            c2 = c2 + rows_v[r0 + 1, pl.ds(h1 + 32, 16)]
            c3 = c3 + rows_v[r0 + 1, pl.ds(h1 + 48, 16)]
            b0 = b0 + rows_v[r0 + 2, pl.ds(h2, 16)]
            b1 = b1 + rows_v[r0 + 2, pl.ds(h2 + 16, 16)]
            b2 = b2 + rows_v[r0 + 2, pl.ds(h2 + 32, 16)]
            b3 = b3 + rows_v[r0 + 2, pl.ds(h2 + 48, 16)]
            c0 = c0 + rows_v[r0 + 3, pl.ds(h3, 16)]
            c1 = c1 + rows_v[r0 + 3, pl.ds(h3 + 16, 16)]
            c2 = c2 + rows_v[r0 + 3, pl.ds(h3 + 32, 16)]
            c3 = c3 + rows_v[r0 + 3, pl.ds(h3 + 48, 16)]
            return (b0, b1, b2, b3, c0, c1, c2, c3)

        b0, b1, b2, b3, c0, c1, c2, c3 = lax.fori_loop(
            0, SEQ // 4, row_body, (z, z, z, z, z, z, z, z))
        stage_v[k, pl.ds(0, 16)] = (b0 + c0) * inv
        stage_v[k, pl.ds(16, 16)] = (b1 + c1) * inv
        stage_v[k, pl.ds(32, 16)] = (b2 + c2) * inv
        stage_v[k, pl.ds(48, 16)] = (b3 + c3) * inv

    fire(0, bufs[0], sems[0])

    def group_body(g8, carry):
        gbase = g8 * _GRP
        for k in range(_GRP):
            g = gbase + k
            nxt = g + 1

            @pl.when(nxt < _BPW)
            def _():
                fire(nxt, bufs[(k + 1) % 2], sems[(k + 1) % 2])

            drain(g, bufs[k % 2], sems[k % 2])
            accum(g, k, bufs[k % 2])
        pltpu.sync_copy(stage_v, out_hbm.at[pl.ds(base0 + gbase, _GRP), :])
        return carry

    lax.fori_loop(0, _NGRP, group_body, 0)


def _sc_pool(x, emb2):
    mesh = plsc.VectorSubcoreMesh(
        core_axis_name="c", subcore_axis_name="s",
        num_cores=_NC, num_subcores=_NS,
    )
    f = pl.kernel(
        _pool_body,
        out_type=jax.ShapeDtypeStruct((BATCH, EMBED), jnp.float32),
        mesh=mesh,
        scratch_types=[
            pltpu.VMEM((SEQ, _BPW), jnp.int32),
            pltpu.VMEM((_BPW * SEQ,), jnp.int32),
            pltpu.VMEM((_BPW * SEQ + 16,), jnp.int32),
            pltpu.VMEM((SEQ, 2 * EMBED), jnp.float32),
            pltpu.VMEM((SEQ, 2 * EMBED), jnp.float32),
            pltpu.VMEM((_GRP, EMBED), jnp.float32),
            pltpu.SemaphoreType.DMA,
            pltpu.SemaphoreType.DMA,
        ],
        compiler_params=pltpu.CompilerParams(
            use_tc_tiling_on_sc=True, needs_layout_passes=False),
    )
    return f(x, emb2)


def _mlp_body(p_ref, w1_ref, b1_ref, w2_ref, b2_ref, out_ref):
    p = p_ref[...]
    h = jnp.dot(p, w1_ref[...], preferred_element_type=jnp.float32) + b1_ref[...]
    z = jnp.dot(h, w2_ref[...], preferred_element_type=jnp.float32) + b2_ref[...]
    m = jnp.max(z, axis=-1, keepdims=True)
    e = jnp.exp(z - m)
    out_ref[...] = e / jnp.sum(e, axis=-1, keepdims=True)


def _tc_mlp(pooled, w1t, b1, w2t, b2):
    return pl.pallas_call(
        _mlp_body,
        out_shape=jax.ShapeDtypeStruct((BATCH, 2), jnp.float32),
    )(pooled, w1t, b1, w2t, b2)


@jax.jit
def kernel(x, emb_table, fc1_w, fc1_b, fc2_w, fc2_b):
    emb2 = emb_table.reshape(VOCAB // 2, 2 * EMBED)
    pooled = _sc_pool(x, emb2)
    return _tc_mlp(
        pooled,
        fc1_w.T,
        fc1_b.reshape(1, 10),
        fc2_w.T,
        fc2_b.reshape(1, 2),
    )
